# baseline (speedup 1.0000x reference)
import functools

import jax
import jax.numpy as jnp
from jax import lax
from jax.experimental import pallas as pl
from jax.experimental.pallas import tpu as pltpu
from jax.experimental.pallas import tpu_sc as plsc

N = 100000
D = 128
G = 1024
OUT = 8

NW = 32
CHUNK = 128
NCHUNK = 25
ROWS_W = CHUNK * NCHUNK
NPAD = NW * ROWS_W


_BM = 3200


def _mm_body(x_ref, w_ref, h_ref):
    i = pl.program_id(0)
    h = jnp.dot(x_ref[...], w_ref[...], preferred_element_type=jnp.float32)
    rows = i * _BM + lax.broadcasted_iota(jnp.int32, (_BM, OUT), 0)
    h_ref[...] = jnp.where(rows < N, h, 0.0)


def _matmul(x, w):
    return pl.pallas_call(
        _mm_body,
        grid=(NPAD // _BM,),
        in_specs=[
            pl.BlockSpec((_BM, D), lambda i: (i, 0)),
            pl.BlockSpec((D, OUT), lambda i: (0, 0)),
        ],
        out_specs=pl.BlockSpec((_BM, OUT), lambda i: (i, 0)),
        out_shape=jax.ShapeDtypeStruct((NPAD, OUT), jnp.float32),
    )(x, w)


_ZROWS = G // 16


def _sc_body(h_hbm, seg_hbm, zero_hbm, out_hbm, acc_sh, segv, hv):
    c = lax.axis_index("c")
    s = lax.axis_index("s")
    w = c * 16 + s

    pltpu.sync_copy(
        zero_hbm.at[pl.ds(s * _ZROWS, _ZROWS), :],
        acc_sh.at[pl.ds(s * _ZROWS, _ZROWS), :],
    )
    plsc.subcore_barrier()

    pltpu.sync_copy(seg_hbm.at[w], segv)
    pltpu.sync_copy(h_hbm.at[w], hv)
    for j in range(NCHUNK):
        pltpu.sync_copy(
            hv.at[pl.ds(j * CHUNK, CHUNK), :],
            acc_sh.at[segv.at[j]],
            add=True,
        )
    plsc.subcore_barrier()

    @pl.when(s == 0)
    def _():
        pltpu.sync_copy(acc_sh, out_hbm.at[c])


def _segsum(h_pad, seg_pad):
    mesh = plsc.VectorSubcoreMesh(core_axis_name="c", subcore_axis_name="s")
    fn = functools.partial(
        pl.kernel,
        mesh=mesh,
        out_type=jax.ShapeDtypeStruct((2, G, OUT), jnp.float32),
        scratch_types=[
            pltpu.VMEM_SHARED((G, OUT), jnp.float32),
            pltpu.VMEM((NCHUNK, CHUNK), jnp.int32),
            pltpu.VMEM((ROWS_W, OUT), jnp.float32),
        ],
        compiler_params=pltpu.CompilerParams(use_tc_tiling_on_sc=False),
    )(_sc_body)
    return fn(
        h_pad.reshape(NW, ROWS_W, OUT),
        seg_pad.reshape(NW, NCHUNK, CHUNK),
        jnp.zeros((G, OUT), jnp.float32),
    )


def _fin_body(p_ref, o_ref):
    pred = p_ref[0] + p_ref[1]
    a = jax.lax.broadcast_in_dim(pred[:, 0:1], (G, OUT), (0, 1))
    b = jax.lax.broadcast_in_dim(pred[:, 1:2], (G, OUT), (0, 1))
    ab = a * b
    col = lax.broadcasted_iota(jnp.int32, (G, OUT), 1)
    o_ref[...] = jnp.where(col == 0, ab, jnp.where(col == 1, -ab, pred))


def _finalize(partial):
    return pl.pallas_call(
        _fin_body,
        out_shape=jax.ShapeDtypeStruct((G, OUT), jnp.float32),
    )(partial)


_P = 256
_Q = 4


def _fused_body(x_ref, sl_ref, ss_ref, w_ref, out_ref, acc_ref):
    i = pl.program_id(0)
    h = jnp.dot(x_ref[...], w_ref[...], preferred_element_type=jnp.float32)
    rows = i * _BM + lax.broadcasted_iota(jnp.int32, (_BM, OUT), 0)
    h = jnp.where(rows < N, h, 0.0)

    seg_lane = sl_ref[0, 0, :]
    seg_sub = ss_ref[...]
    hi_lane = seg_lane >> 2
    lo_sub = seg_sub & 3

    oh = (lax.broadcasted_iota(jnp.int32, (_P, _BM), 0)
          == hi_lane[None, :]).astype(jnp.float32)
    h4 = jnp.concatenate([h, h, h, h], axis=1)
    qcol = lax.broadcasted_iota(jnp.int32, (_BM, 32), 1) >> 3
    bq = jnp.where(lo_sub == qcol, h4, 0.0)
    contrib = jnp.dot(oh, bq, preferred_element_type=jnp.float32,
                      precision=lax.Precision.HIGHEST)

    @pl.when(i == 0)
    def _():
        acc_ref[...] = jnp.zeros_like(acc_ref)

    acc_ref[...] += contrib

    @pl.when(i == pl.num_programs(0) - 1)
    def _():
        acc = acc_ref[...]
        r32 = lax.broadcasted_iota(jnp.int32, (32, 32), 0)
        c32 = lax.broadcasted_iota(jnp.int32, (32, 32), 1)
        grp = (r32 >> 3) == (c32 >> 3)
        sel0 = jnp.where(grp & ((r32 & 7) == 0), 1.0, 0.0)
        sel1 = jnp.where(grp & ((r32 & 7) == 1), 1.0, 0.0)
        a = jnp.dot(acc, sel0, preferred_element_type=jnp.float32)
        b = jnp.dot(acc, sel1, preferred_element_type=jnp.float32)
        ab = a * b
        cmod = lax.broadcasted_iota(jnp.int32, (_P, 32), 1) & 7
        out_ref[...] = jnp.where(cmod == 0, ab, jnp.where(cmod == 1, -ab, acc))


def _fused(x, seg_lane, seg_sub, w):
    return pl.pallas_call(
        _fused_body,
        grid=(NPAD // _BM,),
        in_specs=[
            pl.BlockSpec((_BM, D), lambda i: (i, 0)),
            pl.BlockSpec((1, 1, _BM), lambda i: (i, 0, 0)),
            pl.BlockSpec((_BM, 1), lambda i: (i, 0)),
            pl.BlockSpec((D, OUT), lambda i: (0, 0)),
        ],
        out_specs=pl.BlockSpec((_P, 32), lambda i: (0, 0)),
        out_shape=jax.ShapeDtypeStruct((_P, 32), jnp.float32),
        scratch_shapes=[pltpu.VMEM((_P, 32), jnp.float32)],
    )(x, seg_lane, seg_sub, w)


def kernel(x, segment_ids, W):
    seg = segment_ids.astype(jnp.int32)
    seg_pad = jnp.pad(seg, (0, NPAD - N))
    o2 = _fused(
        x,
        seg_pad.reshape(NPAD // _BM, 1, _BM),
        seg_pad.reshape(NPAD, 1),
        W,
    )
    return o2.reshape(G, OUT)


# speedup vs baseline: 1.2768x; 1.2768x over previous
import functools

import jax
import jax.numpy as jnp
from jax import lax
from jax.experimental import pallas as pl
from jax.experimental.pallas import tpu as pltpu
from jax.experimental.pallas import tpu_sc as plsc

N = 100000
D = 128
G = 1024
OUT = 8

NW = 32
CHUNK = 128
NCHUNK = 25
ROWS_W = CHUNK * NCHUNK
NPAD = NW * ROWS_W


_BM = 3200


def _mm_body(x_ref, w_ref, h_ref):
    i = pl.program_id(0)
    h = jnp.dot(x_ref[...], w_ref[...], preferred_element_type=jnp.float32)
    rows = i * _BM + lax.broadcasted_iota(jnp.int32, (_BM, OUT), 0)
    h_ref[...] = jnp.where(rows < N, h, 0.0)


def _matmul(x, w):
    return pl.pallas_call(
        _mm_body,
        grid=(NPAD // _BM,),
        in_specs=[
            pl.BlockSpec((_BM, D), lambda i: (i, 0)),
            pl.BlockSpec((D, OUT), lambda i: (0, 0)),
        ],
        out_specs=pl.BlockSpec((_BM, OUT), lambda i: (i, 0)),
        out_shape=jax.ShapeDtypeStruct((NPAD, OUT), jnp.float32),
    )(x, w)


_ZROWS = G // 16


def _sc_body(h_hbm, seg_hbm, zero_hbm, out_hbm, acc_sh, segv, hv):
    c = lax.axis_index("c")
    s = lax.axis_index("s")
    w = c * 16 + s

    pltpu.sync_copy(
        zero_hbm.at[pl.ds(s * _ZROWS, _ZROWS), :],
        acc_sh.at[pl.ds(s * _ZROWS, _ZROWS), :],
    )
    plsc.subcore_barrier()

    pltpu.sync_copy(seg_hbm.at[w], segv)
    pltpu.sync_copy(h_hbm.at[w], hv)
    for j in range(NCHUNK):
        pltpu.sync_copy(
            hv.at[pl.ds(j * CHUNK, CHUNK), :],
            acc_sh.at[segv.at[j]],
            add=True,
        )
    plsc.subcore_barrier()

    @pl.when(s == 0)
    def _():
        pltpu.sync_copy(acc_sh, out_hbm.at[c])


def _segsum(h_pad, seg_pad):
    mesh = plsc.VectorSubcoreMesh(core_axis_name="c", subcore_axis_name="s")
    fn = functools.partial(
        pl.kernel,
        mesh=mesh,
        out_type=jax.ShapeDtypeStruct((2, G, OUT), jnp.float32),
        scratch_types=[
            pltpu.VMEM_SHARED((G, OUT), jnp.float32),
            pltpu.VMEM((NCHUNK, CHUNK), jnp.int32),
            pltpu.VMEM((ROWS_W, OUT), jnp.float32),
        ],
        compiler_params=pltpu.CompilerParams(use_tc_tiling_on_sc=False),
    )(_sc_body)
    return fn(
        h_pad.reshape(NW, ROWS_W, OUT),
        seg_pad.reshape(NW, NCHUNK, CHUNK),
        jnp.zeros((G, OUT), jnp.float32),
    )


def _fin_body(p_ref, o_ref):
    pred = p_ref[0] + p_ref[1]
    a = jax.lax.broadcast_in_dim(pred[:, 0:1], (G, OUT), (0, 1))
    b = jax.lax.broadcast_in_dim(pred[:, 1:2], (G, OUT), (0, 1))
    ab = a * b
    col = lax.broadcasted_iota(jnp.int32, (G, OUT), 1)
    o_ref[...] = jnp.where(col == 0, ab, jnp.where(col == 1, -ab, pred))


def _finalize(partial):
    return pl.pallas_call(
        _fin_body,
        out_shape=jax.ShapeDtypeStruct((G, OUT), jnp.float32),
    )(partial)


_P = 256
_Q = 4


def _fused_body(x_ref, sl_ref, ss_ref, w_ref, out_ref, acc_ref):
    i = pl.program_id(0)
    h = jnp.dot(x_ref[...], w_ref[...], preferred_element_type=jnp.float32)
    rows = i * _BM + lax.broadcasted_iota(jnp.int32, (_BM, OUT), 0)
    h = jnp.where(rows < N, h, 0.0)

    seg_lane = sl_ref[0, 0, :]
    seg_sub = ss_ref[...]
    hi_lane = seg_lane >> 2
    lo_sub = seg_sub & 3

    oh = (lax.broadcasted_iota(jnp.int32, (_P, _BM), 0)
          == hi_lane[None, :]).astype(jnp.bfloat16)
    h4 = jnp.concatenate([h, h, h, h], axis=1)
    qcol = lax.broadcasted_iota(jnp.int32, (_BM, 32), 1) >> 3
    bq = jnp.where(lo_sub == qcol, h4, 0.0)
    bq_hi = bq.astype(jnp.bfloat16)
    bq_lo = (bq - bq_hi.astype(jnp.float32)).astype(jnp.bfloat16)
    contrib = (jnp.dot(oh, bq_hi, preferred_element_type=jnp.float32)
               + jnp.dot(oh, bq_lo, preferred_element_type=jnp.float32))

    @pl.when(i == 0)
    def _():
        acc_ref[...] = jnp.zeros_like(acc_ref)

    acc_ref[...] += contrib

    @pl.when(i == pl.num_programs(0) - 1)
    def _():
        acc = acc_ref[...]
        a = jnp.concatenate(
            [jnp.broadcast_to(acc[:, k * 8:k * 8 + 1], (_P, 8)) for k in range(4)],
            axis=1)
        b = jnp.concatenate(
            [jnp.broadcast_to(acc[:, k * 8 + 1:k * 8 + 2], (_P, 8)) for k in range(4)],
            axis=1)
        ab = a * b
        cmod = lax.broadcasted_iota(jnp.int32, (_P, 32), 1) & 7
        out_ref[...] = jnp.where(cmod == 0, ab, jnp.where(cmod == 1, -ab, acc))


def _fused(x, seg_lane, seg_sub, w):
    return pl.pallas_call(
        _fused_body,
        grid=(NPAD // _BM,),
        in_specs=[
            pl.BlockSpec((_BM, D), lambda i: (i, 0)),
            pl.BlockSpec((1, 1, _BM), lambda i: (i, 0, 0)),
            pl.BlockSpec((_BM, 1), lambda i: (i, 0)),
            pl.BlockSpec((D, OUT), lambda i: (0, 0)),
        ],
        out_specs=pl.BlockSpec((_P, 32), lambda i: (0, 0)),
        out_shape=jax.ShapeDtypeStruct((_P, 32), jnp.float32),
        scratch_shapes=[pltpu.VMEM((_P, 32), jnp.float32)],
    )(x, seg_lane, seg_sub, w)


def kernel(x, segment_ids, W):
    seg = segment_ids.astype(jnp.int32)
    seg_pad = jnp.pad(seg, (0, NPAD - N))
    o2 = _fused(
        x,
        seg_pad.reshape(NPAD // _BM, 1, _BM),
        seg_pad.reshape(NPAD, 1),
        W,
    )
    return o2.reshape(G, OUT)


# speedup vs baseline: 1.3043x; 1.0215x over previous
import functools

import jax
import jax.numpy as jnp
from jax import lax
from jax.experimental import pallas as pl
from jax.experimental.pallas import tpu as pltpu
from jax.experimental.pallas import tpu_sc as plsc

N = 100000
D = 128
G = 1024
OUT = 8

NW = 32
CHUNK = 128
NCHUNK = 25
ROWS_W = CHUNK * NCHUNK
NPAD = NW * ROWS_W


_BM = 3200


def _mm_body(x_ref, w_ref, h_ref):
    i = pl.program_id(0)
    h = jnp.dot(x_ref[...], w_ref[...], preferred_element_type=jnp.float32)
    rows = i * _BM + lax.broadcasted_iota(jnp.int32, (_BM, OUT), 0)
    h_ref[...] = jnp.where(rows < N, h, 0.0)


def _matmul(x, w):
    return pl.pallas_call(
        _mm_body,
        grid=(NPAD // _BM,),
        in_specs=[
            pl.BlockSpec((_BM, D), lambda i: (i, 0)),
            pl.BlockSpec((D, OUT), lambda i: (0, 0)),
        ],
        out_specs=pl.BlockSpec((_BM, OUT), lambda i: (i, 0)),
        out_shape=jax.ShapeDtypeStruct((NPAD, OUT), jnp.float32),
    )(x, w)


_ZROWS = G // 16


def _sc_body(h_hbm, seg_hbm, zero_hbm, out_hbm, acc_sh, segv, hv):
    c = lax.axis_index("c")
    s = lax.axis_index("s")
    w = c * 16 + s

    pltpu.sync_copy(
        zero_hbm.at[pl.ds(s * _ZROWS, _ZROWS), :],
        acc_sh.at[pl.ds(s * _ZROWS, _ZROWS), :],
    )
    plsc.subcore_barrier()

    pltpu.sync_copy(seg_hbm.at[w], segv)
    pltpu.sync_copy(h_hbm.at[w], hv)
    for j in range(NCHUNK):
        pltpu.sync_copy(
            hv.at[pl.ds(j * CHUNK, CHUNK), :],
            acc_sh.at[segv.at[j]],
            add=True,
        )
    plsc.subcore_barrier()

    @pl.when(s == 0)
    def _():
        pltpu.sync_copy(acc_sh, out_hbm.at[c])


def _segsum(h_pad, seg_pad):
    mesh = plsc.VectorSubcoreMesh(core_axis_name="c", subcore_axis_name="s")
    fn = functools.partial(
        pl.kernel,
        mesh=mesh,
        out_type=jax.ShapeDtypeStruct((2, G, OUT), jnp.float32),
        scratch_types=[
            pltpu.VMEM_SHARED((G, OUT), jnp.float32),
            pltpu.VMEM((NCHUNK, CHUNK), jnp.int32),
            pltpu.VMEM((ROWS_W, OUT), jnp.float32),
        ],
        compiler_params=pltpu.CompilerParams(use_tc_tiling_on_sc=False),
    )(_sc_body)
    return fn(
        h_pad.reshape(NW, ROWS_W, OUT),
        seg_pad.reshape(NW, NCHUNK, CHUNK),
        jnp.zeros((G, OUT), jnp.float32),
    )


def _fin_body(p_ref, o_ref):
    pred = p_ref[0] + p_ref[1]
    a = jax.lax.broadcast_in_dim(pred[:, 0:1], (G, OUT), (0, 1))
    b = jax.lax.broadcast_in_dim(pred[:, 1:2], (G, OUT), (0, 1))
    ab = a * b
    col = lax.broadcasted_iota(jnp.int32, (G, OUT), 1)
    o_ref[...] = jnp.where(col == 0, ab, jnp.where(col == 1, -ab, pred))


def _finalize(partial):
    return pl.pallas_call(
        _fin_body,
        out_shape=jax.ShapeDtypeStruct((G, OUT), jnp.float32),
    )(partial)


_P = 256
_Q = 4


def _fused_body(x_ref, sl_ref, ss_ref, w_ref, out_ref, acc_ref):
    i = pl.program_id(0)
    h = jnp.dot(x_ref[...], w_ref[...], preferred_element_type=jnp.float32)
    rows = i * _BM + lax.broadcasted_iota(jnp.int32, (_BM, OUT), 0)
    h = jnp.where(rows < N, h, 0.0)

    seg_lane = sl_ref[0, 0, :]
    seg_sub = ss_ref[...]
    hi_lane = seg_lane >> 2
    lo_sub = seg_sub & 3

    iota16 = lax.broadcasted_iota(jnp.int16, (_P, _BM), 0)
    hi16 = hi_lane.astype(jnp.int16)
    oh = jnp.where(iota16 == hi16[None, :],
                   jnp.bfloat16(1), jnp.bfloat16(0))
    h4 = jnp.concatenate([h, h, h, h], axis=1)
    qcol = lax.broadcasted_iota(jnp.int32, (_BM, 32), 1) >> 3
    bq = jnp.where(lo_sub == qcol, h4, 0.0)
    bq_hi = bq.astype(jnp.bfloat16)
    bq_lo = (bq - bq_hi.astype(jnp.float32)).astype(jnp.bfloat16)
    contrib = (jnp.dot(oh, bq_hi, preferred_element_type=jnp.float32)
               + jnp.dot(oh, bq_lo, preferred_element_type=jnp.float32))

    @pl.when(i == 0)
    def _():
        acc_ref[...] = jnp.zeros_like(acc_ref)

    acc_ref[...] += contrib

    @pl.when(i == pl.num_programs(0) - 1)
    def _():
        acc = acc_ref[...]
        a = jnp.concatenate(
            [jnp.broadcast_to(acc[:, k * 8:k * 8 + 1], (_P, 8)) for k in range(4)],
            axis=1)
        b = jnp.concatenate(
            [jnp.broadcast_to(acc[:, k * 8 + 1:k * 8 + 2], (_P, 8)) for k in range(4)],
            axis=1)
        ab = a * b
        cmod = lax.broadcasted_iota(jnp.int32, (_P, 32), 1) & 7
        out_ref[...] = jnp.where(cmod == 0, ab, jnp.where(cmod == 1, -ab, acc))


def _fused(x, seg_lane, seg_sub, w):
    return pl.pallas_call(
        _fused_body,
        grid=(NPAD // _BM,),
        in_specs=[
            pl.BlockSpec((_BM, D), lambda i: (i, 0)),
            pl.BlockSpec((1, 1, _BM), lambda i: (i, 0, 0)),
            pl.BlockSpec((_BM, 1), lambda i: (i, 0)),
            pl.BlockSpec((D, OUT), lambda i: (0, 0)),
        ],
        out_specs=pl.BlockSpec((_P, 32), lambda i: (0, 0)),
        out_shape=jax.ShapeDtypeStruct((_P, 32), jnp.float32),
        scratch_shapes=[pltpu.VMEM((_P, 32), jnp.float32)],
    )(x, seg_lane, seg_sub, w)


def kernel(x, segment_ids, W):
    seg = segment_ids.astype(jnp.int32)
    seg_pad = jnp.pad(seg, (0, NPAD - N))
    o2 = _fused(
        x,
        seg_pad.reshape(NPAD // _BM, 1, _BM),
        seg_pad.reshape(NPAD, 1),
        W,
    )
    return o2.reshape(G, OUT)


# speedup vs baseline: 1.7315x; 1.3275x over previous
import functools

import jax
import jax.numpy as jnp
from jax import lax
from jax.experimental import pallas as pl
from jax.experimental.pallas import tpu as pltpu
from jax.experimental.pallas import tpu_sc as plsc

N = 100000
D = 128
G = 1024
OUT = 8

NW = 32
CHUNK = 128
NCHUNK = 25
ROWS_W = CHUNK * NCHUNK
NPAD = NW * ROWS_W


_BM = 3200


def _mm_body(x_ref, w_ref, h_ref):
    i = pl.program_id(0)
    h = jnp.dot(x_ref[...], w_ref[...], preferred_element_type=jnp.float32)
    rows = i * _BM + lax.broadcasted_iota(jnp.int32, (_BM, OUT), 0)
    h_ref[...] = jnp.where(rows < N, h, 0.0)


def _matmul(x, w):
    return pl.pallas_call(
        _mm_body,
        grid=(NPAD // _BM,),
        in_specs=[
            pl.BlockSpec((_BM, D), lambda i: (i, 0)),
            pl.BlockSpec((D, OUT), lambda i: (0, 0)),
        ],
        out_specs=pl.BlockSpec((_BM, OUT), lambda i: (i, 0)),
        out_shape=jax.ShapeDtypeStruct((NPAD, OUT), jnp.float32),
    )(x, w)


_ZROWS = G // 16


def _sc_body(h_hbm, seg_hbm, zero_hbm, out_hbm, acc_sh, segv, hv):
    c = lax.axis_index("c")
    s = lax.axis_index("s")
    w = c * 16 + s

    pltpu.sync_copy(
        zero_hbm.at[pl.ds(s * _ZROWS, _ZROWS), :],
        acc_sh.at[pl.ds(s * _ZROWS, _ZROWS), :],
    )
    plsc.subcore_barrier()

    pltpu.sync_copy(seg_hbm.at[w], segv)
    pltpu.sync_copy(h_hbm.at[w], hv)
    for j in range(NCHUNK):
        pltpu.sync_copy(
            hv.at[pl.ds(j * CHUNK, CHUNK), :],
            acc_sh.at[segv.at[j]],
            add=True,
        )
    plsc.subcore_barrier()

    @pl.when(s == 0)
    def _():
        pltpu.sync_copy(acc_sh, out_hbm.at[c])


def _segsum(h_pad, seg_pad):
    mesh = plsc.VectorSubcoreMesh(core_axis_name="c", subcore_axis_name="s")
    fn = functools.partial(
        pl.kernel,
        mesh=mesh,
        out_type=jax.ShapeDtypeStruct((2, G, OUT), jnp.float32),
        scratch_types=[
            pltpu.VMEM_SHARED((G, OUT), jnp.float32),
            pltpu.VMEM((NCHUNK, CHUNK), jnp.int32),
            pltpu.VMEM((ROWS_W, OUT), jnp.float32),
        ],
        compiler_params=pltpu.CompilerParams(use_tc_tiling_on_sc=False),
    )(_sc_body)
    return fn(
        h_pad.reshape(NW, ROWS_W, OUT),
        seg_pad.reshape(NW, NCHUNK, CHUNK),
        jnp.zeros((G, OUT), jnp.float32),
    )


def _fin_body(p_ref, o_ref):
    pred = p_ref[0] + p_ref[1]
    a = jax.lax.broadcast_in_dim(pred[:, 0:1], (G, OUT), (0, 1))
    b = jax.lax.broadcast_in_dim(pred[:, 1:2], (G, OUT), (0, 1))
    ab = a * b
    col = lax.broadcasted_iota(jnp.int32, (G, OUT), 1)
    o_ref[...] = jnp.where(col == 0, ab, jnp.where(col == 1, -ab, pred))


def _finalize(partial):
    return pl.pallas_call(
        _fin_body,
        out_shape=jax.ShapeDtypeStruct((G, OUT), jnp.float32),
    )(partial)


_P = 64
_Q = 16
_FBM = 5000


def _fused_body(x_ref, sl_ref, ss_ref, w16_ref, out_ref, acc_ref):
    i = pl.program_id(0)
    h16 = jnp.dot(x_ref[...], w16_ref[...],
                  preferred_element_type=jnp.float32)

    hi16 = (sl_ref[0, 0, :] >> 4).astype(jnp.int16)
    lo32 = ss_ref[...] & 15

    iota_p = lax.broadcasted_iota(jnp.int16, (_P, _FBM), 0)
    oh = jnp.where(iota_p == hi16[None, :],
                   jnp.bfloat16(1), jnp.bfloat16(0))

    qcol = lax.broadcasted_iota(jnp.int32, (_FBM, 128), 1) >> 3
    h_bf = h16.astype(jnp.bfloat16)
    bq = jnp.where(lo32 == qcol, h_bf, jnp.bfloat16(0))
    contrib = jnp.dot(oh, bq, preferred_element_type=jnp.float32)

    @pl.when(i == 0)
    def _():
        acc_ref[...] = jnp.zeros_like(acc_ref)

    acc_ref[...] += contrib

    @pl.when(i == pl.num_programs(0) - 1)
    def _():
        acc = acc_ref[...]
        a = jnp.concatenate(
            [jnp.broadcast_to(acc[:, k * 8:k * 8 + 1], (_P, 8))
             for k in range(_Q)], axis=1)
        b = jnp.concatenate(
            [jnp.broadcast_to(acc[:, k * 8 + 1:k * 8 + 2], (_P, 8))
             for k in range(_Q)], axis=1)
        ab = a * b
        cmod = lax.broadcasted_iota(jnp.int32, (_P, 128), 1) & 7
        out_ref[...] = jnp.where(cmod == 0, ab, jnp.where(cmod == 1, -ab, acc))


def _fused(x, seg_lane, seg_sub, w16):
    return pl.pallas_call(
        _fused_body,
        grid=(N // _FBM,),
        in_specs=[
            pl.BlockSpec((_FBM, D), lambda i: (i, 0)),
            pl.BlockSpec((1, 1, _FBM), lambda i: (i, 0, 0)),
            pl.BlockSpec((_FBM, 1), lambda i: (i, 0)),
            pl.BlockSpec((D, 128), lambda i: (0, 0)),
        ],
        out_specs=pl.BlockSpec((_P, 128), lambda i: (0, 0)),
        out_shape=jax.ShapeDtypeStruct((_P, 128), jnp.float32),
        scratch_shapes=[pltpu.VMEM((_P, 128), jnp.float32)],
    )(x, seg_lane, seg_sub, w16)


def kernel(x, segment_ids, W):
    seg = segment_ids.astype(jnp.int32)
    w16 = jnp.tile(W, (1, _Q))
    o2 = _fused(
        x,
        seg.reshape(N // _FBM, 1, _FBM),
        seg.reshape(N, 1),
        w16,
    )
    return o2.reshape(G, OUT)


# speedup vs baseline: 1.8976x; 1.0960x over previous
import functools

import jax
import jax.numpy as jnp
from jax import lax
from jax.experimental import pallas as pl
from jax.experimental.pallas import tpu as pltpu
from jax.experimental.pallas import tpu_sc as plsc

N = 100000
D = 128
G = 1024
OUT = 8

NW = 32
CHUNK = 128
NCHUNK = 25
ROWS_W = CHUNK * NCHUNK
NPAD = NW * ROWS_W


_BM = 3200


def _mm_body(x_ref, w_ref, h_ref):
    i = pl.program_id(0)
    h = jnp.dot(x_ref[...], w_ref[...], preferred_element_type=jnp.float32)
    rows = i * _BM + lax.broadcasted_iota(jnp.int32, (_BM, OUT), 0)
    h_ref[...] = jnp.where(rows < N, h, 0.0)


def _matmul(x, w):
    return pl.pallas_call(
        _mm_body,
        grid=(NPAD // _BM,),
        in_specs=[
            pl.BlockSpec((_BM, D), lambda i: (i, 0)),
            pl.BlockSpec((D, OUT), lambda i: (0, 0)),
        ],
        out_specs=pl.BlockSpec((_BM, OUT), lambda i: (i, 0)),
        out_shape=jax.ShapeDtypeStruct((NPAD, OUT), jnp.float32),
    )(x, w)


_ZROWS = G // 16


def _sc_body(h_hbm, seg_hbm, zero_hbm, out_hbm, acc_sh, segv, hv):
    c = lax.axis_index("c")
    s = lax.axis_index("s")
    w = c * 16 + s

    pltpu.sync_copy(
        zero_hbm.at[pl.ds(s * _ZROWS, _ZROWS), :],
        acc_sh.at[pl.ds(s * _ZROWS, _ZROWS), :],
    )
    plsc.subcore_barrier()

    pltpu.sync_copy(seg_hbm.at[w], segv)
    pltpu.sync_copy(h_hbm.at[w], hv)
    for j in range(NCHUNK):
        pltpu.sync_copy(
            hv.at[pl.ds(j * CHUNK, CHUNK), :],
            acc_sh.at[segv.at[j]],
            add=True,
        )
    plsc.subcore_barrier()

    @pl.when(s == 0)
    def _():
        pltpu.sync_copy(acc_sh, out_hbm.at[c])


def _segsum(h_pad, seg_pad):
    mesh = plsc.VectorSubcoreMesh(core_axis_name="c", subcore_axis_name="s")
    fn = functools.partial(
        pl.kernel,
        mesh=mesh,
        out_type=jax.ShapeDtypeStruct((2, G, OUT), jnp.float32),
        scratch_types=[
            pltpu.VMEM_SHARED((G, OUT), jnp.float32),
            pltpu.VMEM((NCHUNK, CHUNK), jnp.int32),
            pltpu.VMEM((ROWS_W, OUT), jnp.float32),
        ],
        compiler_params=pltpu.CompilerParams(use_tc_tiling_on_sc=False),
    )(_sc_body)
    return fn(
        h_pad.reshape(NW, ROWS_W, OUT),
        seg_pad.reshape(NW, NCHUNK, CHUNK),
        jnp.zeros((G, OUT), jnp.float32),
    )


def _fin_body(p_ref, o_ref):
    pred = p_ref[0] + p_ref[1]
    a = jax.lax.broadcast_in_dim(pred[:, 0:1], (G, OUT), (0, 1))
    b = jax.lax.broadcast_in_dim(pred[:, 1:2], (G, OUT), (0, 1))
    ab = a * b
    col = lax.broadcasted_iota(jnp.int32, (G, OUT), 1)
    o_ref[...] = jnp.where(col == 0, ab, jnp.where(col == 1, -ab, pred))


def _finalize(partial):
    return pl.pallas_call(
        _fin_body,
        out_shape=jax.ShapeDtypeStruct((G, OUT), jnp.float32),
    )(partial)


_P = 64
_Q = 16
_FBM = 10000


def _fused_body(x_ref, sl_ref, ss_ref, w16_ref, out_ref, acc_ref):
    i = pl.program_id(0)
    h16 = jnp.dot(x_ref[...], w16_ref[...],
                  preferred_element_type=jnp.float32)

    hi16 = (sl_ref[0, 0, :] >> 4).astype(jnp.int16)
    lo32 = ss_ref[...] & 15

    iota_p = lax.broadcasted_iota(jnp.int16, (_P, _FBM), 0)
    oh = jnp.where(iota_p == hi16[None, :],
                   jnp.bfloat16(1), jnp.bfloat16(0))

    qcol = lax.broadcasted_iota(jnp.int32, (_FBM, 128), 1) >> 3
    h_bf = h16.astype(jnp.bfloat16)
    bq = jnp.where(lo32 == qcol, h_bf, jnp.bfloat16(0))
    contrib = jnp.dot(oh, bq, preferred_element_type=jnp.float32)

    @pl.when(i == 0)
    def _():
        acc_ref[...] = jnp.zeros_like(acc_ref)

    acc_ref[...] += contrib

    @pl.when(i == pl.num_programs(0) - 1)
    def _():
        acc = acc_ref[...]
        a = jnp.concatenate(
            [jnp.broadcast_to(acc[:, k * 8:k * 8 + 1], (_P, 8))
             for k in range(_Q)], axis=1)
        b = jnp.concatenate(
            [jnp.broadcast_to(acc[:, k * 8 + 1:k * 8 + 2], (_P, 8))
             for k in range(_Q)], axis=1)
        ab = a * b
        cmod = lax.broadcasted_iota(jnp.int32, (_P, 128), 1) & 7
        out_ref[...] = jnp.where(cmod == 0, ab, jnp.where(cmod == 1, -ab, acc))


def _fused(x, seg_lane, seg_sub, w16):
    return pl.pallas_call(
        _fused_body,
        grid=(N // _FBM,),
        in_specs=[
            pl.BlockSpec((_FBM, D), lambda i: (i, 0)),
            pl.BlockSpec((1, 1, _FBM), lambda i: (i, 0, 0)),
            pl.BlockSpec((_FBM, 1), lambda i: (i, 0)),
            pl.BlockSpec((D, 128), lambda i: (0, 0)),
        ],
        out_specs=pl.BlockSpec((_P, 128), lambda i: (0, 0)),
        out_shape=jax.ShapeDtypeStruct((_P, 128), jnp.float32),
        scratch_shapes=[pltpu.VMEM((_P, 128), jnp.float32)],
    )(x, seg_lane, seg_sub, w16)


def kernel(x, segment_ids, W):
    seg = segment_ids.astype(jnp.int32)
    w16 = jnp.tile(W, (1, _Q))
    o2 = _fused(
        x,
        seg.reshape(N // _FBM, 1, _FBM),
        seg.reshape(N, 1),
        w16,
    )
    return o2.reshape(G, OUT)
